# baseline (speedup 1.0000x reference)
import functools

import jax
import jax.numpy as jnp
from jax import lax
from jax.experimental import pallas as pl
from jax.experimental.pallas import tpu as pltpu
from jax.experimental.pallas import tpu_sc as plsc

NUM_CORES = 2
NUM_SUBCORES = 16
NUM_WORKERS = NUM_CORES * NUM_SUBCORES

BATCH = 4096
SEQ = 50
DIM = 64
TOTAL = BATCH * SEQ
PER_WORKER = TOTAL // NUM_WORKERS

CHUNK = 800
NCHUNK = PER_WORKER // CHUNK
NBUF = 2


def _gather_body(table_hbm, idx_hbm, out_hbm, idx_v, rows_v, gsem, psem):
    wid = lax.axis_index("s") * NUM_CORES + lax.axis_index("c")
    base = wid * PER_WORKER
    pltpu.sync_copy(idx_hbm.at[pl.ds(base, PER_WORKER)], idx_v)

    gathers = [None] * NBUF
    puts = [None] * NBUF
    for c in range(NCHUNK + 1):
        if c < NCHUNK:
            s = c % NBUF
            if puts[s] is not None:
                puts[s].wait()
            gathers[s] = pltpu.make_async_copy(
                table_hbm.at[idx_v.at[pl.ds(c * CHUNK, CHUNK)]],
                rows_v.at[s],
                gsem.at[s],
            )
            gathers[s].start()
        if c >= 1:
            sp = (c - 1) % NBUF
            gathers[sp].wait()
            puts[sp] = pltpu.make_async_copy(
                rows_v.at[sp],
                out_hbm.at[pl.ds(base + (c - 1) * CHUNK, CHUNK)],
                psem.at[sp],
            )
            puts[sp].start()
    for s in range(NBUF):
        if puts[s] is not None:
            puts[s].wait()


@jax.jit
def _sc_gather(inputs_flat, table):
    mesh = plsc.VectorSubcoreMesh(core_axis_name="c", subcore_axis_name="s")
    run = functools.partial(
        pl.kernel,
        out_type=jax.ShapeDtypeStruct((TOTAL, DIM), jnp.float32),
        mesh=mesh,
        scratch_types=[
            pltpu.VMEM((PER_WORKER,), jnp.int32),
            pltpu.VMEM((NBUF, CHUNK, DIM), jnp.float32),
            pltpu.SemaphoreType.DMA((NBUF,)),
            pltpu.SemaphoreType.DMA((NBUF,)),
        ],
        compiler_params=pltpu.CompilerParams(use_tc_tiling_on_sc=False),
    )(_gather_body)
    return run(table, inputs_flat)


def kernel(inputs, table):
    out = _sc_gather(inputs.reshape(TOTAL), table)
    return out.reshape(BATCH, SEQ, DIM)


# speedup vs baseline: 1.3289x; 1.3289x over previous
import functools

import jax
import jax.numpy as jnp
from jax import lax
from jax.experimental import pallas as pl
from jax.experimental.pallas import tpu as pltpu
from jax.experimental.pallas import tpu_sc as plsc

NUM_CORES = 2
NUM_SUBCORES = 16
NUM_WORKERS = NUM_CORES * NUM_SUBCORES

BATCH = 4096
SEQ = 50
DIM = 64
TOTAL = BATCH * SEQ
PER_WORKER = TOTAL // NUM_WORKERS

CHUNK = 400
NCHUNK = PER_WORKER // CHUNK
NBUF = 2


def _gather_body(table_hbm, idx_hbm, out_hbm, idx_v, rows_v, gsem, psem):
    wid = lax.axis_index("s") * NUM_CORES + lax.axis_index("c")
    base = wid * PER_WORKER
    pltpu.sync_copy(idx_hbm.at[pl.ds(base, PER_WORKER)], idx_v)

    puts = [None] * NBUF
    for c in range(NCHUNK):
        s = c % NBUF
        if puts[s] is not None:
            puts[s].wait()
        buf = rows_v.at[s]
        off = c * CHUNK

        @pl.loop(0, CHUNK // 16)
        def _row(g, off=off, buf=buf, s=s):
            vec = idx_v[pl.ds(off + g * 16, 16)]
            for l in range(16):
                pltpu.make_async_copy(
                    table_hbm.at[pl.ds(vec[l], 1)],
                    buf.at[pl.ds(g * 16 + l, 1)],
                    gsem.at[s],
                ).start()

        @pl.loop(0, CHUNK, unroll=8)
        def _drain(j, buf=buf, s=s):
            pltpu.make_async_copy(
                table_hbm.at[pl.ds(0, 1)], buf.at[pl.ds(0, 1)], gsem.at[s]
            ).wait()
        puts[s] = pltpu.make_async_copy(
            buf, out_hbm.at[pl.ds(base + off, CHUNK)], psem.at[s]
        )
        puts[s].start()
    for s in range(NBUF):
        if puts[s] is not None:
            puts[s].wait()


@jax.jit
def _sc_gather(inputs_flat, table):
    mesh = plsc.VectorSubcoreMesh(core_axis_name="c", subcore_axis_name="s")
    run = functools.partial(
        pl.kernel,
        out_type=jax.ShapeDtypeStruct((TOTAL, DIM), jnp.float32),
        mesh=mesh,
        scratch_types=[
            pltpu.VMEM((PER_WORKER,), jnp.int32),
            pltpu.VMEM((NBUF, CHUNK, DIM), jnp.float32),
            pltpu.SemaphoreType.DMA((NBUF,)),
            pltpu.SemaphoreType.DMA((NBUF,)),
        ],
        compiler_params=pltpu.CompilerParams(use_tc_tiling_on_sc=True),
    )(_gather_body)
    return run(table, inputs_flat)


def kernel(inputs, table):
    out = _sc_gather(inputs.reshape(TOTAL), table)
    return out.reshape(BATCH, SEQ, DIM)


# speedup vs baseline: 1.9857x; 1.4942x over previous
import jax
import jax.numpy as jnp
from jax import lax
from jax.experimental import pallas as pl
from jax.experimental.pallas import tpu as pltpu
from jax.experimental.pallas import tpu_sc as plsc

NUM_CORES = 2
NUM_SUBCORES = 16
NUM_WORKERS = NUM_CORES * NUM_SUBCORES

BATCH = 4096
SEQ = 50
DIM = 64
TOTAL = BATCH * SEQ
PER_WORKER = TOTAL // NUM_WORKERS
B_PER_W = BATCH // NUM_WORKERS

CB = 8
CHUNK = CB * SEQ
NCHUNK = B_PER_W // CB
NBUF = 2


def _gather_body(table_hbm, idx_hbm, out_hbm, idx_v, rows_v, gsem, psem):
    wid = lax.axis_index("s") * NUM_CORES + lax.axis_index("c")
    base = wid * PER_WORKER
    b_lo = wid * B_PER_W
    pltpu.sync_copy(idx_hbm.at[pl.ds(base, PER_WORKER)], idx_v)

    puts = [None] * NBUF
    for c in range(NCHUNK):
        s = c % NBUF
        if puts[s] is not None:
            puts[s].wait()
        buf = rows_v.at[s]
        off = c * CHUNK

        @pl.loop(0, CHUNK // 16)
        def _row(g, off=off, buf=buf, s=s):
            vec = idx_v[pl.ds(off + g * 16, 16)]
            for l in range(16):
                j = g * 16 + l
                pltpu.make_async_copy(
                    table_hbm.at[pl.ds(vec[l], 1), pl.ds(0, 1)],
                    buf.at[pl.ds(j // SEQ, 1), pl.ds(j % SEQ, 1)],
                    gsem.at[s],
                ).start()

        @pl.loop(0, CHUNK, unroll=8)
        def _drain(j, buf=buf, s=s):
            pltpu.make_async_copy(
                table_hbm.at[pl.ds(0, 1), pl.ds(0, 1)],
                buf.at[pl.ds(0, 1), pl.ds(0, 1)],
                gsem.at[s],
            ).wait()

        puts[s] = pltpu.make_async_copy(
            rows_v.at[s], out_hbm.at[pl.ds(b_lo + c * CB, CB)], psem.at[s]
        )
        puts[s].start()
    for s in range(NBUF):
        if puts[s] is not None:
            puts[s].wait()


@jax.jit
def _sc_gather(inputs_flat, table3):
    mesh = plsc.VectorSubcoreMesh(core_axis_name="c", subcore_axis_name="s")
    run = pl.kernel(
        _gather_body,
        out_type=jax.ShapeDtypeStruct((BATCH, SEQ, DIM), jnp.float32),
        mesh=mesh,
        scratch_types=[
            pltpu.VMEM((PER_WORKER,), jnp.int32),
            pltpu.VMEM((NBUF, CB, SEQ, DIM), jnp.float32),
            pltpu.SemaphoreType.DMA((NBUF,)),
            pltpu.SemaphoreType.DMA((NBUF,)),
        ],
        compiler_params=pltpu.CompilerParams(use_tc_tiling_on_sc=True),
    )
    return run(table3, inputs_flat)


def kernel(inputs, table):
    return _sc_gather(inputs.reshape(TOTAL), table.reshape(1000000, 1, DIM))
